# Initial kernel scaffold; baseline (speedup 1.0000x reference)
import jax
import jax.numpy as jnp
from jax.experimental import pallas as pl


def kernel(x, w0, b0, w1, b1, w2, b2, w3, b3, w8, b8, w9, b9, w11, b11):
    raise NotImplementedError("write your pallas kernel here")


# speedup vs baseline: 1.0557x; 1.0557x over previous
import functools

import jax
import jax.numpy as jnp
from jax import lax
from jax.experimental import pallas as pl
from jax.experimental.pallas import tpu as pltpu


def _sppcspc_bf16_kernel(H, W,
                         x_ref,
                         w0_ref, b0_ref, w1_ref, b1_ref,
                         w2_ref, b2_ref, w3_ref, b3_ref,
                         w8_ref, b8_ref, w9_ref, b9_ref,
                         w11_ref, b11_ref,
                         o_ref):
    HW = H * W
    x = x_ref[...].astype(jnp.bfloat16)

    pos = lax.broadcasted_iota(jnp.int32, (1, HW), 1)
    ww = jnp.bitwise_and(pos, W - 1) if (W & (W - 1)) == 0 else lax.rem(pos, W)

    def rot(a, s):
        amt = (-s) % HW
        return a if amt == 0 else pltpu.roll(a, amt, 1)

    def h_valid(dh):
        return (pos >= -dh * W) & (pos < HW - dh * W)

    def w_valid(dw):
        return (ww + dw >= 0) & (ww + dw < W)

    def silu_q(y):
        return (y * jax.nn.sigmoid(y)).astype(jnp.bfloat16)

    def cbs1x1(a, w_r, b_r):
        y = jnp.dot(w_r[...], a, preferred_element_type=jnp.float32) + b_r[...]
        return silu_q(y)

    def cbs3x3(a, w_r, b_r):
        cout = b_r.shape[0]
        acc = jnp.zeros((cout, HW), jnp.float32)
        zero = jnp.zeros((), jnp.bfloat16)
        for kh in range(3):
            for kw in range(3):
                dh, dw = kh - 1, kw - 1
                tap = jnp.where(h_valid(dh) & w_valid(dw), rot(a, dh * W + dw),
                                zero)
                acc = acc + jnp.dot(w_r[kh * 3 + kw], tap,
                                    preferred_element_type=jnp.float32)
        return silu_q(acc + b_r[...])

    ninf = jnp.array(-jnp.inf, jnp.bfloat16)

    def maxpool_r2(a):
        r = a
        for dw in (-2, -1, 1, 2):
            r = jnp.maximum(r, jnp.where(w_valid(dw), rot(a, dw), ninf))
        out = r
        for dh in (-2, -1, 1, 2):
            out = jnp.maximum(out, jnp.where(h_valid(dh), rot(r, dh * W), ninf))
        return out

    x0 = cbs1x1(x, w0_ref, b0_ref)
    x1 = cbs1x1(x, w1_ref, b1_ref)
    x1 = cbs3x3(x1, w2_ref, b2_ref)
    x1 = cbs1x1(x1, w3_ref, b3_ref)

    p5 = maxpool_r2(x1)
    p9 = maxpool_r2(p5)
    p13 = maxpool_r2(p9)

    y = (jnp.dot(w8_ref[0], x1, preferred_element_type=jnp.float32)
         + jnp.dot(w8_ref[1], p5, preferred_element_type=jnp.float32)
         + jnp.dot(w8_ref[2], p9, preferred_element_type=jnp.float32)
         + jnp.dot(w8_ref[3], p13, preferred_element_type=jnp.float32)
         + b8_ref[...])
    y = silu_q(y)

    y = cbs3x3(y, w9_ref, b9_ref)

    out = (jnp.dot(w11_ref[0], y, preferred_element_type=jnp.float32)
           + jnp.dot(w11_ref[1], x0, preferred_element_type=jnp.float32)
           + b11_ref[...])
    o_ref[...] = (out * jax.nn.sigmoid(out)).astype(o_ref.dtype)


@jax.jit
def _sppcspc_forward(x_nchw, *weights):
    N, C, H, W = x_nchw.shape
    HW = H * W
    x3 = x_nchw.reshape(N, C, HW)
    n_out = weights[-1].shape[0]

    def const_spec(a):
        nd = a.ndim
        return pl.BlockSpec(a.shape, lambda n: (0,) * nd)

    kern = functools.partial(_sppcspc_bf16_kernel, H, W)
    out3 = pl.pallas_call(
        kern,
        out_shape=jax.ShapeDtypeStruct((N, n_out, HW), jnp.float32),
        grid=(N,),
        in_specs=[pl.BlockSpec((None, C, HW), lambda n: (n, 0, 0))]
                 + [const_spec(w) for w in weights],
        out_specs=pl.BlockSpec((None, n_out, HW), lambda n: (n, 0, 0)),
        compiler_params=pltpu.CompilerParams(dimension_semantics=("parallel",)),
    )(x3, *weights)
    return out3.reshape(N, n_out, H, W)


def kernel(x, w0, b0, w1, b1, w2, b2, w3, b3, w8, b8, w9, b9, w11, b11):
    bf = jnp.bfloat16
    f32 = jnp.float32
    ws = (w0.astype(bf), b0.astype(f32),
          w1.astype(bf), b1.astype(f32),
          w2.astype(bf), b2.astype(f32),
          w3.astype(bf), b3.astype(f32),
          w8.astype(bf), b8.astype(f32),
          w9.astype(bf), b9.astype(f32),
          w11.astype(bf), b11.astype(f32))
    return _sppcspc_forward(x, *ws)


# speedup vs baseline: 1.1454x; 1.0849x over previous
import functools

import jax
import jax.numpy as jnp
from jax import lax
from jax.experimental import pallas as pl
from jax.experimental.pallas import tpu as pltpu

_G = 72


def _sppcspc_kernel(H, W,
                    x_ref,
                    w01_ref, b01_ref, w2_ref, b2_ref, w3_ref, b3_ref,
                    w8_ref, b8_ref, w9_ref, b9_ref, w11_ref, b11_ref,
                    o_ref,
                    s_conv, s_x0, s_x1, s_r, s_p5, s_p9):
    HW = H * W
    f32 = jnp.float32
    bf16 = jnp.bfloat16

    row = lax.broadcasted_iota(jnp.int32, (HW, 1), 0)
    ww = jnp.bitwise_and(row, W - 1)

    def w_valid(dw):
        return (ww + dw >= 0) & (ww + dw < W)

    zero = jnp.zeros((), bf16)
    ninf = jnp.array(-jnp.inf, bf16)

    s_conv[:_G, :] = jnp.full((_G, s_conv.shape[1]), 0.0, bf16)
    s_conv[_G + HW:, :] = jnp.full((_G, s_conv.shape[1]), 0.0, bf16)
    s_r[:_G, :] = jnp.full((_G, s_r.shape[1]), -jnp.inf, bf16)
    s_r[_G + HW:, :] = jnp.full((_G, s_r.shape[1]), -jnp.inf, bf16)

    def silu_q(y):
        return (y * jax.nn.sigmoid(y)).astype(bf16)

    def shifted(ref, off):
        return ref[pl.ds(_G + off, HW), :]

    def wshift(val, dw):
        return pltpu.roll(val, (-dw) % HW, 0)

    def cbs3x3(center, src_ref, w_r, b_r):
        rows = {-1: shifted(src_ref, -W), 0: center, 1: shifted(src_ref, W)}
        acc = jnp.dot(center, w_r[4], preferred_element_type=f32)
        for kh in range(3):
            for kw in range(3):
                dh, dw = kh - 1, kw - 1
                if dh == 0 and dw == 0:
                    continue
                if dw == 0:
                    tap = rows[dh]
                else:
                    tap = jnp.where(w_valid(dw), wshift(rows[dh], dw), zero)
                acc = acc + jnp.dot(tap, w_r[kh * 3 + kw],
                                    preferred_element_type=f32)
        return silu_q(acc + b_r[...])

    def maxpool_r2(center, dst_ref):
        r = center
        for dw in (-2, -1, 1, 2):
            r = jnp.maximum(r, jnp.where(w_valid(dw), wshift(center, dw),
                                         ninf))
        s_r[pl.ds(_G, HW), :] = r
        out = r
        for dh in (-2, -1, 1, 2):
            out = jnp.maximum(out, shifted(s_r, dh * W))
        if dst_ref is not None:
            dst_ref[...] = out
        return out

    xb = x_ref[...].astype(bf16)

    h01 = lax.dot_general(xb, w01_ref[...], (((0,), (0,)), ((), ())),
                          preferred_element_type=f32) + b01_ref[...]
    s01 = silu_q(h01)
    half = s01.shape[1] // 2
    x0 = s01[:, :half]
    s_x0[...] = x0
    t1 = s01[:, half:]
    s_conv[pl.ds(_G, HW), :] = t1

    t2 = cbs3x3(t1, s_conv, w2_ref, b2_ref)

    x1 = silu_q(jnp.dot(t2, w3_ref[...], preferred_element_type=f32)
                + b3_ref[...])
    s_x1[...] = x1

    p5 = maxpool_r2(x1, s_p5)
    p9 = maxpool_r2(p5, s_p9)
    p13 = maxpool_r2(p9, None)

    y = (jnp.dot(s_x1[...], w8_ref[0], preferred_element_type=f32)
         + jnp.dot(s_p5[...], w8_ref[1], preferred_element_type=f32)
         + jnp.dot(s_p9[...], w8_ref[2], preferred_element_type=f32)
         + jnp.dot(p13, w8_ref[3], preferred_element_type=f32)
         + b8_ref[...])
    y = silu_q(y)
    s_conv[pl.ds(_G, HW), :] = y

    y2 = cbs3x3(y, s_conv, w9_ref, b9_ref)

    out = (lax.dot_general(w11_ref[0], y2, (((1,), (1,)), ((), ())),
                           preferred_element_type=f32)
           + lax.dot_general(w11_ref[1], s_x0[...], (((1,), (1,)), ((), ())),
                             preferred_element_type=f32)
           + b11_ref[...])
    o_ref[...] = (out * jax.nn.sigmoid(out)).astype(o_ref.dtype)


@jax.jit
def _sppcspc_forward(x_nchw, *weights):
    N, C, H, W = x_nchw.shape
    HW = H * W
    x3 = x_nchw.reshape(N, C, HW)
    n_out = weights[-2].shape[1]
    half = weights[-2].shape[2]

    def const_spec(a):
        nd = a.ndim
        return pl.BlockSpec(a.shape, lambda n: (0,) * nd)

    kern = functools.partial(_sppcspc_kernel, H, W)
    out3 = pl.pallas_call(
        kern,
        out_shape=jax.ShapeDtypeStruct((N, n_out, HW), jnp.float32),
        grid=(N,),
        in_specs=[pl.BlockSpec((None, C, HW), lambda n: (n, 0, 0))]
                 + [const_spec(w) for w in weights],
        out_specs=pl.BlockSpec((None, n_out, HW), lambda n: (n, 0, 0)),
        scratch_shapes=[
            pltpu.VMEM((HW + 2 * _G, half), jnp.bfloat16),
            pltpu.VMEM((HW, half), jnp.bfloat16),
            pltpu.VMEM((HW, half), jnp.bfloat16),
            pltpu.VMEM((HW + 2 * _G, half), jnp.bfloat16),
            pltpu.VMEM((HW, half), jnp.bfloat16),
            pltpu.VMEM((HW, half), jnp.bfloat16),
        ],
        compiler_params=pltpu.CompilerParams(dimension_semantics=("parallel",)),
    )(x3, *weights)
    return out3.reshape(N, n_out, H, W)


def kernel(x, w0, b0, w1, b1, w2, b2, w3, b3, w8, b8, w9, b9, w11, b11):
    bf = jnp.bfloat16
    f32 = jnp.float32
    w01 = jnp.concatenate([w0, w1], axis=0).T.astype(bf)
    b01 = jnp.concatenate([b0, b1], axis=0).reshape(1, -1).astype(f32)
    ws = (w01, b01,
          jnp.transpose(w2, (0, 2, 1)).astype(bf), b2.reshape(1, -1).astype(f32),
          w3.T.astype(bf), b3.reshape(1, -1).astype(f32),
          jnp.transpose(w8, (0, 2, 1)).astype(bf), b8.reshape(1, -1).astype(f32),
          jnp.transpose(w9, (0, 2, 1)).astype(bf), b9.reshape(1, -1).astype(f32),
          w11.astype(bf), b11.astype(f32))
    return _sppcspc_forward(x, *ws)


# speedup vs baseline: 1.2653x; 1.1047x over previous
import functools

import jax
import jax.numpy as jnp
from jax import lax
from jax.experimental import pallas as pl
from jax.experimental.pallas import tpu as pltpu

_GC = 40
_GP = 192


def _sppcspc_kernel(H, W,
                    x_ref,
                    w01_ref, b01_ref, w2_ref, b2_ref, w3_ref, b3_ref,
                    w8_ref, b8_ref, w9_ref, b9_ref, w11_ref, b11_ref,
                    o_ref,
                    s_conv, s_x0, s_x1, s_a5, s_a9, s_a13):
    HW = H * W
    f32 = jnp.float32
    bf16 = jnp.bfloat16

    row = lax.broadcasted_iota(jnp.int32, (HW, 1), 0)
    ww = jnp.bitwise_and(row, W - 1)

    def w_valid(dw):
        return (ww + dw >= 0) & (ww + dw < W)

    ninf = jnp.array(-jnp.inf, bf16)

    s_conv[:_GC, :] = jnp.full((_GC, s_conv.shape[1]), 0.0, bf16)
    s_conv[_GC + HW:, :] = jnp.full((_GC, s_conv.shape[1]), 0.0, bf16)
    for s in (s_a5, s_a9, s_a13):
        s[:_GP, :] = jnp.full((_GP, s.shape[1]), -jnp.inf, bf16)
        s[_GP + HW:, :] = jnp.full((_GP, s.shape[1]), -jnp.inf, bf16)

    def silu_q(y):
        return (y * jax.nn.sigmoid(y)).astype(bf16)

    def wshift(val, dw):
        return pltpu.roll(val, (-dw) % HW, 0)

    def cbs3x3(center, w_r, b_r):
        half = center.shape[1]
        rc = jnp.concatenate(
            [s_conv[pl.ds(_GC - W, HW), :], center,
             s_conv[pl.ds(_GC + W, HW), :]], axis=1)
        u = jnp.dot(rc, w_r[...], preferred_element_type=f32)
        acc = (u[:, half:2 * half]
               + jnp.where(w_valid(-1), wshift(u[:, :half], -1), 0.0)
               + jnp.where(w_valid(1), wshift(u[:, 2 * half:], 1), 0.0))
        return silu_q(acc + b_r[...])

    def wmax(center, pieces):
        r = center
        for dw, val in pieces:
            r = jnp.maximum(r, jnp.where(w_valid(dw), wshift(val, dw), ninf))
        return r

    def hmax(center, src_ref, radius):
        out = center
        for dh in range(-radius, radius + 1):
            if dh != 0:
                out = jnp.maximum(out, src_ref[pl.ds(_GP + dh * W, HW), :])
        return out

    xb = x_ref[...].astype(bf16)

    h01 = lax.dot_general(xb, w01_ref[...], (((0,), (0,)), ((), ())),
                          preferred_element_type=f32) + b01_ref[...]
    s01 = silu_q(h01)
    half = s01.shape[1] // 2
    s_x0[...] = s01[:, :half]
    t1 = s01[:, half:]
    s_conv[pl.ds(_GC, HW), :] = t1

    t2 = cbs3x3(t1, w2_ref, b2_ref)

    x1 = silu_q(jnp.dot(t2, w3_ref[...], preferred_element_type=f32)
                + b3_ref[...])
    s_x1[...] = x1

    a5 = wmax(x1, [(-2, x1), (-1, x1), (1, x1), (2, x1)])
    s_a5[pl.ds(_GP, HW), :] = a5
    a9 = wmax(a5, [(-2, a5), (2, a5)])
    s_a9[pl.ds(_GP, HW), :] = a9
    a13 = wmax(a9, [(-4, a5), (4, a5)])
    s_a13[pl.ds(_GP, HW), :] = a13

    p5 = hmax(a5, s_a5, 2)
    p9 = hmax(a9, s_a9, 4)
    p13 = hmax(a13, s_a13, 6)

    y = (jnp.dot(s_x1[...], w8_ref[0], preferred_element_type=f32)
         + jnp.dot(p5, w8_ref[1], preferred_element_type=f32)
         + jnp.dot(p9, w8_ref[2], preferred_element_type=f32)
         + jnp.dot(p13, w8_ref[3], preferred_element_type=f32)
         + b8_ref[...])
    y = silu_q(y)
    s_conv[pl.ds(_GC, HW), :] = y

    y2 = cbs3x3(y, w9_ref, b9_ref)

    out = (lax.dot_general(w11_ref[0], y2, (((1,), (1,)), ((), ())),
                           preferred_element_type=f32)
           + lax.dot_general(w11_ref[1], s_x0[...], (((1,), (1,)), ((), ())),
                             preferred_element_type=f32)
           + b11_ref[...])
    o_ref[...] = (out * jax.nn.sigmoid(out)).astype(o_ref.dtype)


@jax.jit
def _sppcspc_forward(x_nchw, *weights):
    N, C, H, W = x_nchw.shape
    HW = H * W
    x3 = x_nchw.reshape(N, C, HW)
    n_out = weights[-2].shape[1]
    half = weights[-2].shape[2]

    def const_spec(a):
        nd = a.ndim
        return pl.BlockSpec(a.shape, lambda n: (0,) * nd)

    kern = functools.partial(_sppcspc_kernel, H, W)
    out3 = pl.pallas_call(
        kern,
        out_shape=jax.ShapeDtypeStruct((N, n_out, HW), jnp.float32),
        grid=(N,),
        in_specs=[pl.BlockSpec((None, C, HW), lambda n: (n, 0, 0))]
                 + [const_spec(w) for w in weights],
        out_specs=pl.BlockSpec((None, n_out, HW), lambda n: (n, 0, 0)),
        scratch_shapes=[
            pltpu.VMEM((HW + 2 * _GC, half), jnp.bfloat16),
            pltpu.VMEM((HW, half), jnp.bfloat16),
            pltpu.VMEM((HW, half), jnp.bfloat16),
            pltpu.VMEM((HW + 2 * _GP, half), jnp.bfloat16),
            pltpu.VMEM((HW + 2 * _GP, half), jnp.bfloat16),
            pltpu.VMEM((HW + 2 * _GP, half), jnp.bfloat16),
        ],
        compiler_params=pltpu.CompilerParams(dimension_semantics=("parallel",)),
    )(x3, *weights)
    return out3.reshape(N, n_out, H, W)


def _stack3x3(w):
    return jnp.concatenate(
        [jnp.concatenate([w[kh * 3 + kw].T for kw in range(3)], axis=1)
         for kh in range(3)], axis=0)


def kernel(x, w0, b0, w1, b1, w2, b2, w3, b3, w8, b8, w9, b9, w11, b11):
    bf = jnp.bfloat16
    f32 = jnp.float32
    w01 = jnp.concatenate([w0, w1], axis=0).T.astype(bf)
    b01 = jnp.concatenate([b0, b1], axis=0).reshape(1, -1).astype(f32)
    ws = (w01, b01,
          _stack3x3(w2).astype(bf), b2.reshape(1, -1).astype(f32),
          w3.T.astype(bf), b3.reshape(1, -1).astype(f32),
          jnp.transpose(w8, (0, 2, 1)).astype(bf), b8.reshape(1, -1).astype(f32),
          _stack3x3(w9).astype(bf), b9.reshape(1, -1).astype(f32),
          w11.astype(bf), b11.astype(f32))
    return _sppcspc_forward(x, *ws)


# speedup vs baseline: 1.3234x; 1.0459x over previous
import functools

import jax
import jax.numpy as jnp
from jax import lax
from jax.experimental import pallas as pl
from jax.experimental.pallas import tpu as pltpu

_GC = 40
_GP = 192


def _sppcspc_kernel(H, W,
                    x_ref,
                    w01_ref, b01_ref, wb_ref, bsm_ref, w11_ref, b11_ref,
                    o_ref,
                    s_conv, s_x0, s_x1, s_a5, s_a9, s_a13):
    HW = H * W
    f32 = jnp.float32
    bf16 = jnp.bfloat16

    row = lax.broadcasted_iota(jnp.int32, (HW, 1), 0)
    ww = jnp.bitwise_and(row, W - 1)

    def w_valid(dw):
        return (ww + dw >= 0) & (ww + dw < W)

    ninf = jnp.array(-jnp.inf, bf16)

    s_conv[:_GC, :] = jnp.full((_GC, s_conv.shape[1]), 0.0, bf16)
    s_conv[_GC + HW:, :] = jnp.full((_GC, s_conv.shape[1]), 0.0, bf16)
    for s in (s_a5, s_a9, s_a13):
        s[:_GP, :] = jnp.full((_GP, s.shape[1]), -jnp.inf, bf16)
        s[_GP + HW:, :] = jnp.full((_GP, s.shape[1]), -jnp.inf, bf16)

    def silu_f(y):
        return y * jax.nn.sigmoid(y)

    def silu_q(y):
        return silu_f(y).astype(bf16)

    def wshift(val, dw):
        return pltpu.roll(val, (-dw) % HW, 0)

    def cbs3x3(center, base, b_r):
        half = center.shape[1]
        w_all = jnp.concatenate(
            [jnp.concatenate([wb_ref[base + kh * 3 + kw] for kw in range(3)],
                             axis=1) for kh in range(3)], axis=0)
        rc = jnp.concatenate(
            [s_conv[pl.ds(_GC - W, HW), :], center,
             s_conv[pl.ds(_GC + W, HW), :]], axis=1)
        u = jnp.dot(rc, w_all, preferred_element_type=f32)
        acc = (u[:, half:2 * half]
               + jnp.where(w_valid(-1), wshift(u[:, :half], -1), 0.0)
               + jnp.where(w_valid(1), wshift(u[:, 2 * half:], 1), 0.0))
        return silu_q(acc + b_r[...])

    def wmax(center, pieces):
        r = center
        for dw, val in pieces:
            r = jnp.maximum(r, jnp.where(w_valid(dw), wshift(val, dw), ninf))
        return r

    def hmax(center, src_ref, radius):
        out = center
        for dh in range(-radius, radius + 1):
            if dh != 0:
                out = jnp.maximum(out, src_ref[pl.ds(_GP + dh * W, HW), :])
        return out

    xb = x_ref[...].astype(bf16)

    h01 = lax.dot_general(xb, w01_ref[...], (((0,), (0,)), ((), ())),
                          preferred_element_type=f32) + b01_ref[...]
    s01 = silu_q(h01)
    half = s01.shape[1] // 2
    s_x0[...] = s01[:, :half]
    t1 = s01[:, half:]
    s_conv[pl.ds(_GC, HW), :] = t1

    t2 = cbs3x3(t1, 0, bsm_ref[0:1, :])

    x1 = silu_q(jnp.dot(t2, wb_ref[22], preferred_element_type=f32)
                + bsm_ref[1:2, :])
    s_x1[...] = x1

    a5 = wmax(x1, [(-2, x1), (-1, x1), (1, x1), (2, x1)])
    s_a5[pl.ds(_GP, HW), :] = a5
    a9 = wmax(a5, [(-2, a5), (2, a5)])
    s_a9[pl.ds(_GP, HW), :] = a9
    a13 = wmax(a9, [(-4, a5), (4, a5)])
    s_a13[pl.ds(_GP, HW), :] = a13

    p5 = hmax(a5, s_a5, 2)
    p9 = hmax(a9, s_a9, 4)
    p13 = hmax(a13, s_a13, 6)

    y = (jnp.dot(s_x1[...], wb_ref[18], preferred_element_type=f32)
         + jnp.dot(p5, wb_ref[19], preferred_element_type=f32)
         + jnp.dot(p9, wb_ref[20], preferred_element_type=f32)
         + jnp.dot(p13, wb_ref[21], preferred_element_type=f32)
         + bsm_ref[2:3, :])
    y = silu_q(y)
    s_conv[pl.ds(_GC, HW), :] = y

    y2 = cbs3x3(y, 9, bsm_ref[3:4, :])

    out = (lax.dot_general(w11_ref[0], y2, (((1,), (1,)), ((), ())),
                           preferred_element_type=f32)
           + lax.dot_general(w11_ref[1], s_x0[...], (((1,), (1,)), ((), ())),
                             preferred_element_type=f32)
           + b11_ref[...])
    o_ref[...] = silu_f(out).astype(o_ref.dtype)


@jax.jit
def _sppcspc_forward(x_nchw, *weights):
    N, C, H, W = x_nchw.shape
    HW = H * W
    x3 = x_nchw.reshape(N, C, HW)
    n_out = weights[-2].shape[1]
    half = weights[-2].shape[2]

    def const_spec(a):
        nd = a.ndim
        return pl.BlockSpec(a.shape, lambda n: (0,) * nd)

    kern = functools.partial(_sppcspc_kernel, H, W)
    out3 = pl.pallas_call(
        kern,
        out_shape=jax.ShapeDtypeStruct((N, n_out, HW), jnp.float32),
        grid=(N,),
        in_specs=[pl.BlockSpec((None, C, HW), lambda n: (n, 0, 0))]
                 + [const_spec(w) for w in weights],
        out_specs=pl.BlockSpec((None, n_out, HW), lambda n: (n, 0, 0)),
        scratch_shapes=[
            pltpu.VMEM((HW + 2 * _GC, half), jnp.bfloat16),
            pltpu.VMEM((HW, half), jnp.bfloat16),
            pltpu.VMEM((HW, half), jnp.bfloat16),
            pltpu.VMEM((HW + 2 * _GP, half), jnp.bfloat16),
            pltpu.VMEM((HW + 2 * _GP, half), jnp.bfloat16),
            pltpu.VMEM((HW + 2 * _GP, half), jnp.bfloat16),
        ],
        compiler_params=pltpu.CompilerParams(dimension_semantics=("parallel",)),
    )(x3, *weights)
    return out3.reshape(N, n_out, H, W)


def kernel(x, w0, b0, w1, b1, w2, b2, w3, b3, w8, b8, w9, b9, w11, b11):
    bf = jnp.bfloat16
    f32 = jnp.float32
    w01 = jnp.concatenate([w0, w1], axis=0).T.astype(bf)
    b01 = jnp.concatenate([b0, b1], axis=0).reshape(1, -1).astype(f32)
    wb = jnp.concatenate([w2, w9, w8, w3[None]], axis=0)
    wb = jnp.transpose(wb, (0, 2, 1)).astype(bf)
    bsm = jnp.concatenate([b2, b3, b8, b9], axis=1).T.astype(f32)
    ws = (w01, b01, wb, bsm, w11.astype(bf), b11.astype(f32))
    return _sppcspc_forward(x, *ws)
